# baseline (speedup 1.0000x reference)
import functools

import jax
import jax.numpy as jnp
import numpy as np
from jax import lax
from jax.experimental import pallas as pl
from jax.experimental.pallas import tpu as pltpu
from jax.experimental.pallas import tpu_sc as plsc

VOCAB = 1000000
D = 64
B = 4096
L = 200
SCALE = 8.0

NC = 2
NS = 16
NW = NC * NS
BBLK = B // NW
TOTAL = B * L
XROWS = TOTAL // 128
XR_PER_W = XROWS // NW
OPAD = 133

NFULL = VOCAB // 128
NTAIL = VOCAB - NFULL * 128
ITERS_A = (NFULL + NW - 1) // NW


def _tr_body(tT_hbm, tail_hbm, out_hbm, sb0, sb1, tb0, tb1, stail, rs0, rs1, ws0, ws1):
    c = lax.axis_index("c")
    s = lax.axis_index("s")
    w = s * NC + c
    lanes = lax.iota(jnp.int32, 16)
    sbufs, tbufs = (sb0, sb1), (tb0, tb1)
    rsems, wsems = (rs0, rs1), (ws0, ws1)
    vvecs = [sg * 16 + lanes for sg in range(8)]

    def rd(cblk, par):
        @pl.when(cblk < NFULL)
        def _():
            for rb in range(8):
                pltpu.async_copy(
                    tT_hbm.at[pl.ds(rb * 8, 8), pl.ds(cblk * 128, 128)],
                    sbufs[par].at[pl.ds(rb * 8, 8), :],
                    rsems[par],
                )

    rd(w, 0)

    def it_body(i, carry):
        for par in range(2):
            it = i * 2 + par
            cblk = w + NW * it
            sb, tb = sbufs[par], tbufs[par]

            @pl.when(cblk < NFULL)
            def _():
                pltpu.make_async_copy(
                    out_hbm.at[pl.ds(0, D), :], sb, rsems[par]
                ).wait()

            rd(w + NW * (it + 1), 1 - par)

            @pl.when(cblk < NFULL)
            def _():
                @pl.when(it >= 2)
                def _():
                    pltpu.make_async_copy(
                        tb.at[:, pl.ds(0, 128)],
                        out_hbm.at[pl.ds(cblk * 128, 128), :],
                        wsems[par],
                    ).wait()

                @plsc.parallel_loop(0, D, unroll=4)
                def _(d):
                    dcol = jnp.full((16,), d, jnp.int32)
                    for sg in range(8):
                        v = sb[d, pl.ds(sg * 16, 16)]
                        plsc.store_scatter(tb, [vvecs[sg], dcol], v)

                pltpu.async_copy(
                    tb.at[:, pl.ds(0, 128)],
                    out_hbm.at[pl.ds(cblk * 128, 128), :],
                    wsems[par],
                )
        return carry

    lax.fori_loop(0, (ITERS_A + 1) // 2, it_body, 0)

    for par in range(2):
        last = w + NW * (ITERS_A - 2 + par)

        @pl.when(last < NFULL)
        def _():
            pltpu.make_async_copy(
                tbufs[par].at[:, pl.ds(0, 128)],
                out_hbm.at[pl.ds(last * 128, 128), :],
                wsems[par],
            ).wait()

    @pl.when(w == 0)
    def _():
        pltpu.sync_copy(tail_hbm, stail)

        @plsc.parallel_loop(0, D, unroll=4)
        def _(d):
            dcol = jnp.full((16,), d, jnp.int32)
            for sg in range(NTAIL // 16):
                v = stail[d, pl.ds(sg * 16, 16)]
                plsc.store_scatter(tb0, [vvecs[sg], dcol], v)

        pltpu.sync_copy(
            tb0.at[pl.ds(0, NTAIL), pl.ds(0, 128)],
            out_hbm.at[pl.ds(NFULL * 128, NTAIL), :],
        )


def _embed_body(
    x_hbm, tab_hbm, out_hbm,
    xb, pb0, pb1, gb0, gb1, ob0, ob1,
    gs0, gs1, os0, os1,
):
    c = lax.axis_index("c")
    s = lax.axis_index("s")
    w = s * NC + c
    lanes = lax.iota(jnp.int32, 16)
    pbufs, gbufs, obufs = (pb0, pb1), (gb0, gb1), (ob0, ob1)
    gsems, osems = (gs0, gs1), (os0, os1)

    d_hi = [(16 * k + lanes) >> 3 for k in range(4)]
    d_lo = [(16 * k + lanes) & 7 for k in range(4)]

    pltpu.sync_copy(x_hbm.at[pl.ds(w * XR_PER_W, XR_PER_W)], xb)

    def stage(l, pb):
        for sg in range(8):
            t = (sg * 16 + lanes) * L + l
            pb[pl.ds(sg * 16, 16)] = plsc.load_gather(xb, [t >> 7, t & 127])

    def gath(pb, gb, sem):
        for q in range(4):
            pltpu.async_copy(
                tab_hbm.at[pb.at[pl.ds(q * 32, 32)]],
                gb.at[pl.ds(q * 32, 32)],
                sem,
            )

    stage(0, pb0)
    gath(pb0, gb0, gs0)

    def outer(i, carry):
        for par in range(2):
            l = i * 2 + par
            pb, gb, ob = pbufs[par], gbufs[par], obufs[par]
            pltpu.make_async_copy(tab_hbm.at[pl.ds(0, 128)], gb, gsems[par]).wait()

            @pl.when(l < L - 1)
            def _():
                stage(l + 1, pbufs[1 - par])
                gath(pbufs[1 - par], gbufs[1 - par], gsems[1 - par])

            @pl.when(l >= 2)
            def _():
                pltpu.make_async_copy(
                    ob.at[:, :, pl.ds(0, BBLK)], out_hbm.at[l, :, w], osems[par]
                ).wait()

            @plsc.parallel_loop(0, BBLK, unroll=8)
            def _(j):
                col = jnp.full((16,), j, jnp.int32)
                for k in range(4):
                    v = gb[j, pl.ds(k * 16, 16)]
                    plsc.store_scatter(ob, [d_hi[k], d_lo[k], col], v * SCALE)
            for rb in range(8):
                pltpu.async_copy(
                    ob.at[rb, :, pl.ds(0, BBLK)],
                    out_hbm.at[l, rb, w],
                    osems[par],
                )
        return carry

    lax.fori_loop(0, L // 2, outer, 0)
    pltpu.make_async_copy(
        ob0.at[:, :, pl.ds(0, BBLK)], out_hbm.at[L - 2, :, w], os0
    ).wait()
    pltpu.make_async_copy(
        ob1.at[:, :, pl.ds(0, BBLK)], out_hbm.at[L - 1, :, w], os1
    ).wait()


def _mesh():
    return plsc.VectorSubcoreMesh(
        core_axis_name="c", subcore_axis_name="s", num_cores=NC, num_subcores=NS
    )


@jax.jit
def _transpose_table(tT, tail):
    return pl.kernel(
        _tr_body,
        out_type=jax.ShapeDtypeStruct((VOCAB, 128), jnp.float32),
        mesh=_mesh(),
        scratch_types=[
            pltpu.VMEM((D, 128), jnp.float32),
            pltpu.VMEM((D, 128), jnp.float32),
            pltpu.VMEM((128, 129), jnp.float32),
            pltpu.VMEM((128, 129), jnp.float32),
            pltpu.VMEM((D, NTAIL), jnp.float32),
            pltpu.SemaphoreType.DMA,
            pltpu.SemaphoreType.DMA,
            pltpu.SemaphoreType.DMA,
            pltpu.SemaphoreType.DMA,
        ],
        compiler_params=pltpu.CompilerParams(needs_layout_passes=False),
    )(tT, tail)


@jax.jit
def _embed(x2d, tab):
    return pl.kernel(
        _embed_body,
        out_type=jax.ShapeDtypeStruct((L, 8, NW, 8, BBLK), jnp.float32),
        mesh=_mesh(),
        scratch_types=[
            pltpu.VMEM((XR_PER_W, 128), jnp.int32),
            pltpu.VMEM((128,), jnp.int32),
            pltpu.VMEM((128,), jnp.int32),
            pltpu.VMEM((128, 128), jnp.float32),
            pltpu.VMEM((128, 128), jnp.float32),
            pltpu.VMEM((8, 8, OPAD), jnp.float32),
            pltpu.VMEM((8, 8, OPAD), jnp.float32),
            pltpu.SemaphoreType.DMA,
            pltpu.SemaphoreType.DMA,
            pltpu.SemaphoreType.DMA,
            pltpu.SemaphoreType.DMA,
        ],
        compiler_params=pltpu.CompilerParams(needs_layout_passes=False),
    )(x2d, tab)


def kernel(x, table):
    x2d = x.astype(jnp.int32).reshape(XROWS, 128)
    tT = table.T
    tprep = _transpose_table(tT, tT[:, NFULL * 128:])
    out5d = _embed(x2d, tprep)
    return out5d.transpose(2, 4, 0, 1, 3).reshape(B, L, D)


# speedup vs baseline: 1.2731x; 1.2731x over previous
import functools

import jax
import jax.numpy as jnp
import numpy as np
from jax import lax
from jax.experimental import pallas as pl
from jax.experimental.pallas import tpu as pltpu
from jax.experimental.pallas import tpu_sc as plsc

VOCAB = 1000000
D = 64
B = 4096
L = 200
SCALE = 8.0

NC = 2
NS = 16
NW = NC * NS
BBLK = B // NW
TOTAL = B * L
XROWS = TOTAL // 128
XR_PER_W = XROWS // NW
OPAD = 133


def _embed_body(
    x_hbm, tab_hbm, out_hbm,
    xb, pb0, pb1, gb0, gb1, ob0, ob1,
    gs0, gs1, os0, os1,
):
    c = lax.axis_index("c")
    s = lax.axis_index("s")
    w = s * NC + c
    lanes = lax.iota(jnp.int32, 16)
    pbufs, gbufs, obufs = (pb0, pb1), (gb0, gb1), (ob0, ob1)
    gsems, osems = (gs0, gs1), (os0, os1)

    d_hi = [(16 * k + lanes) >> 3 for k in range(4)]
    d_lo = [(16 * k + lanes) & 7 for k in range(4)]

    pltpu.sync_copy(x_hbm.at[pl.ds(w * XR_PER_W, XR_PER_W)], xb)

    def stage(l, pb):
        for sg in range(8):
            t = (sg * 16 + lanes) * L + l
            pb[pl.ds(sg * 16, 16)] = plsc.load_gather(xb, [t >> 7, t & 127])

    stage(0, pb0)
    pltpu.async_copy(tab_hbm.at[pb0], gb0, gs0)

    def outer(i, carry):
        for par in range(2):
            l = i * 2 + par
            pb, gb, ob = pbufs[par], gbufs[par], obufs[par]
            pltpu.make_async_copy(tab_hbm.at[pb], gb, gsems[par]).wait()

            @pl.when(l < L - 1)
            def _():
                stage(l + 1, pbufs[1 - par])
                pltpu.async_copy(
                    tab_hbm.at[pbufs[1 - par]], gbufs[1 - par], gsems[1 - par]
                )

            @pl.when(l >= 2)
            def _():
                pltpu.make_async_copy(
                    ob.at[:, :, pl.ds(0, BBLK)], out_hbm.at[l, :, w], osems[par]
                ).wait()

            @plsc.parallel_loop(0, BBLK, unroll=4)
            def _(j):
                col = jnp.full((16,), j, jnp.int32)
                for k in range(4):
                    v = gb[j, pl.ds(k * 16, 16)]
                    plsc.store_scatter(ob, [d_hi[k], d_lo[k], col], v * SCALE)
            pltpu.async_copy(
                ob.at[:, :, pl.ds(0, BBLK)], out_hbm.at[l, :, w], osems[par]
            )
        return carry

    lax.fori_loop(0, L // 2, outer, 0)
    pltpu.make_async_copy(
        ob0.at[:, :, pl.ds(0, BBLK)], out_hbm.at[L - 2, :, w], os0
    ).wait()
    pltpu.make_async_copy(
        ob1.at[:, :, pl.ds(0, BBLK)], out_hbm.at[L - 1, :, w], os1
    ).wait()


@jax.jit
def _embed(x2d, tpad):
    mesh = plsc.VectorSubcoreMesh(
        core_axis_name="c", subcore_axis_name="s", num_cores=NC, num_subcores=NS
    )
    return pl.kernel(
        _embed_body,
        out_type=jax.ShapeDtypeStruct((L, 8, NW, 8, BBLK), jnp.float32),
        mesh=mesh,
        scratch_types=[
            pltpu.VMEM((XR_PER_W, 128), jnp.int32),
            pltpu.VMEM((128,), jnp.int32),
            pltpu.VMEM((128,), jnp.int32),
            pltpu.VMEM((128, 128), jnp.float32),
            pltpu.VMEM((128, 128), jnp.float32),
            pltpu.VMEM((8, 8, OPAD), jnp.float32),
            pltpu.VMEM((8, 8, OPAD), jnp.float32),
            pltpu.SemaphoreType.DMA,
            pltpu.SemaphoreType.DMA,
            pltpu.SemaphoreType.DMA,
            pltpu.SemaphoreType.DMA,
        ],
        compiler_params=pltpu.CompilerParams(needs_layout_passes=False),
    )(x2d, tpad)


def kernel(x, table):
    x2d = x.astype(jnp.int32).reshape(XROWS, 128)
    tpad = jnp.pad(table, ((0, 0), (0, 128 - D)))
    out5d = _embed(x2d, tpad)
    return out5d.transpose(2, 4, 0, 1, 3).reshape(B, L, D)
